# baseline (speedup 1.0000x reference)
import jax
import jax.numpy as jnp
from jax.experimental import pallas as pl
from jax.experimental.pallas import tpu as pltpu

_HALVES = 2


def _mm_mask_kernel(ea_ref, eb_ref, aa_ref, ab_ref, w_ref, out_ref):
    w = w_ref[...]
    for h, (e_ref, a_ref) in enumerate(((ea_ref, aa_ref), (eb_ref, ab_ref))):
        e = e_ref[0]
        logits = jax.lax.dot_general(
            e, w,
            dimension_numbers=(((1,), (1,)), ((), ())),
            preferred_element_type=jnp.float32,
        )
        att = a_ref[0]
        out_ref[h, 0] = jnp.where(att, logits, -jnp.inf)


def kernel(emb_sentences, att_sentences, W):
    B, L, D = emb_sentences.shape
    NL = W.shape[0]
    att3 = att_sentences.reshape(B, L, 1)
    H = B // _HALVES

    out = pl.pallas_call(
        _mm_mask_kernel,
        grid=(H,),
        in_specs=[
            pl.BlockSpec((1, L, D), lambda i: (i, 0, 0)),
            pl.BlockSpec((1, L, D), lambda i: (i + H, 0, 0)),
            pl.BlockSpec((1, L, 1), lambda i: (i, 0, 0)),
            pl.BlockSpec((1, L, 1), lambda i: (i + H, 0, 0)),
            pl.BlockSpec((NL, D), lambda i: (0, 0)),
        ],
        out_specs=pl.BlockSpec((_HALVES, 1, L, NL), lambda i: (0, i, 0, 0)),
        out_shape=jax.ShapeDtypeStruct((_HALVES, H, L, NL), jnp.float32),
        compiler_params=pltpu.CompilerParams(
            dimension_semantics=("arbitrary",),
        ),
    )(emb_sentences, emb_sentences, att3, att3, W)
    return out.reshape(B, L, NL)


# speedup vs baseline: 1.8276x; 1.8276x over previous
import jax
import jax.numpy as jnp
from jax.experimental import pallas as pl
from jax.experimental.pallas import tpu as pltpu


def _mm_mask_kernel(emb_ref, att_ref, w_ref, out_ref):
    e = emb_ref[0]
    w = w_ref[...]
    logits_t = jax.lax.dot_general(
        w, e,
        dimension_numbers=(((1,), (1,)), ((), ())),
        preferred_element_type=jnp.float32,
    )
    att = att_ref[0]
    out_ref[0] = jnp.where(att, logits_t, -jnp.inf)


def kernel(emb_sentences, att_sentences, W):
    B, L, D = emb_sentences.shape
    NL = W.shape[0]
    att3 = att_sentences.reshape(B, 1, L)

    out = pl.pallas_call(
        _mm_mask_kernel,
        grid=(B,),
        in_specs=[
            pl.BlockSpec((1, L, D), lambda i: (i, 0, 0)),
            pl.BlockSpec((1, 1, L), lambda i: (i, 0, 0)),
            pl.BlockSpec((NL, D), lambda i: (0, 0)),
        ],
        out_specs=pl.BlockSpec((1, NL, L), lambda i: (i, 0, 0)),
        out_shape=jax.ShapeDtypeStruct((B, NL, L), jnp.float32),
        compiler_params=pltpu.CompilerParams(
            dimension_semantics=("parallel",),
        ),
    )(emb_sentences, att3, W)
    return out.transpose(0, 2, 1)
